# Initial kernel scaffold; baseline (speedup 1.0000x reference)
import jax
import jax.numpy as jnp
from jax.experimental import pallas as pl


def kernel(features, edge_idx, W0, b0, W1, b1, W2, b2):
    raise NotImplementedError("write your pallas kernel here")


# speedup vs baseline: 5.2689x; 5.2689x over previous
import jax
import jax.numpy as jnp
from jax import lax
from jax.experimental import pallas as pl
from jax.experimental.pallas import tpu as pltpu
from jax.experimental.pallas import tpu_sc as plsc

N = 10000
C = 128
E = 160000
K_PROP = 10
ALPHA = 0.1

NP = 10240
NC, NS, L = 2, 16, 16
NT = NC * NS
EP = 163840
EPT = EP // NT
CHUNK = 128
NCH = EPT // CHUNK
ER = EP // CHUNK
RPT = NP // NS

_MESH = plsc.VectorSubcoreMesh(
    core_axis_name="c", subcore_axis_name="s", num_cores=NC, num_subcores=NS
)


def _deg_body(srcm, dstm, deg_out, srcp_out, srcv, dstv, srcpv, onesv, degv,
              deg_sh):
    c = lax.axis_index("c")
    s = lax.axis_index("s")
    wid = c * NS + s
    base = wid * NCH

    pltpu.sync_copy(srcm.at[pl.ds(base, NCH)], srcv)
    pltpu.sync_copy(dstm.at[pl.ds(base, NCH)], dstv)

    for r in range(NCH):
        for q in range(CHUNK // L):
            sl = pl.ds(q * L, L)
            sv = srcv[r, sl]
            dv = dstv[r, sl]
            srcpv[r, sl] = jnp.where(sv == dv, jnp.int32(N), sv)
    pltpu.sync_copy(srcpv, srcp_out.at[pl.ds(base, NCH)])

    ones = jnp.ones((L,), jnp.float32)
    init = jnp.where(c == 0, 1.0, 0.0) * ones
    for q in range(CHUNK // L):
        onesv[pl.ds(q * L, L)] = ones
    for r in range(RPT // L):
        degv[pl.ds(r * L, L)] = init
    pltpu.sync_copy(degv, deg_sh.at[pl.ds(s * RPT, RPT)])
    plsc.subcore_barrier()

    for j in range(NCH):
        pltpu.sync_copy(onesv, deg_sh.at[srcpv.at[j]], add=True)
    plsc.subcore_barrier()

    pltpu.sync_copy(deg_sh.at[pl.ds(s * RPT, RPT)],
                    deg_out.at[c, pl.ds(s * RPT, RPT)])


_deg_kernel = pl.kernel(
    _deg_body,
    out_type=(
        jax.ShapeDtypeStruct((NC, NP), jnp.float32),
        jax.ShapeDtypeStruct((ER, CHUNK), jnp.int32),
    ),
    mesh=_MESH,
    scratch_types=(
        pltpu.VMEM((NCH, CHUNK), jnp.int32),
        pltpu.VMEM((NCH, CHUNK), jnp.int32),
        pltpu.VMEM((NCH, CHUNK), jnp.int32),
        pltpu.VMEM((CHUNK,), jnp.float32),
        pltpu.VMEM((RPT,), jnp.float32),
        pltpu.VMEM_SHARED((NP,), jnp.float32),
    ),
)


def _prop_body(y_hbm, srcp, dstm, acc_out, srcv, dstv, rows, zbuf, sem,
               acc_sh):
    c = lax.axis_index("c")
    s = lax.axis_index("s")
    wid = c * NS + s
    base = wid * NCH

    zv = jnp.zeros((L,), jnp.float32)
    for r in range(64):
        for q in range(C // L):
            zbuf[r, pl.ds(q * L, L)] = zv
    for t in range(RPT // 64):
        pltpu.sync_copy(zbuf, acc_sh.at[pl.ds(s * RPT + t * 64, 64)])

    pltpu.sync_copy(srcp.at[pl.ds(base, NCH)], srcv)
    pltpu.sync_copy(dstm.at[pl.ds(base, NCH)], dstv)
    plsc.subcore_barrier()

    for j in range(NCH):
        pltpu.async_copy(y_hbm.at[srcv.at[j]], rows, sem).wait()
        pltpu.sync_copy(rows, acc_sh.at[dstv.at[j]], add=True)
    plsc.subcore_barrier()

    pltpu.sync_copy(acc_sh.at[pl.ds(s * RPT, RPT)],
                    acc_out.at[c, pl.ds(s * RPT, RPT)])


_prop_kernel = pl.kernel(
    _prop_body,
    out_type=jax.ShapeDtypeStruct((NC, NP, C), jnp.float32),
    mesh=_MESH,
    scratch_types=(
        pltpu.VMEM((NCH, CHUNK), jnp.int32),
        pltpu.VMEM((NCH, CHUNK), jnp.int32),
        pltpu.VMEM((CHUNK, C), jnp.float32),
        pltpu.VMEM((64, C), jnp.float32),
        pltpu.SemaphoreType.DMA,
        pltpu.VMEM_SHARED((NP, C), jnp.float32),
    ),
)


def _mlp_body(x_ref, w0_ref, b0_ref, w1_ref, b1_ref, w2_ref, b2_ref, out_ref):
    x = x_ref[...]
    h = lax.dot_general(x, w0_ref[...], (((1,), (1,)), ((), ())),
                        preferred_element_type=jnp.float32) + b0_ref[...]
    h = jnp.maximum(h, 0.0)
    h = lax.dot_general(h, w1_ref[...], (((1,), (1,)), ((), ())),
                        preferred_element_type=jnp.float32) + b1_ref[...]
    h = jnp.maximum(h, 0.0)
    out_ref[...] = lax.dot_general(h, w2_ref[...], (((1,), (1,)), ((), ())),
                                   preferred_element_type=jnp.float32) + b2_ref[...]


def _mlp(features, W0, b0, W1, b1, W2, b2):
    RB = 400
    full = lambda shape: pl.BlockSpec(shape, lambda i: (0,) * len(shape))
    return pl.pallas_call(
        _mlp_body,
        grid=(N // RB,),
        in_specs=[
            pl.BlockSpec((RB, 256), lambda i: (i, 0)),
            full((512, 256)), full((1, 512)),
            full((512, 512)), full((1, 512)),
            full((128, 512)), full((1, 128)),
        ],
        out_specs=pl.BlockSpec((RB, C), lambda i: (i, 0)),
        out_shape=jax.ShapeDtypeStruct((N, C), jnp.float32),
    )(features, W0, b0.reshape(1, 512), W1, b1.reshape(1, 512),
      W2, b2.reshape(1, 128))


def _prep_body(deg_ref, logits_ref, y0_ref, c2b_ref, clb_ref, cb_ref):
    d = deg_ref[0] + deg_ref[1]
    pid = pl.program_id(0)
    row = lax.broadcasted_iota(jnp.int32, d.shape, 0) + pid * d.shape[0]
    cc = jnp.where(row < N, lax.rsqrt(jnp.maximum(d, 1e-20)), 0.0)
    lg = logits_ref[...]
    y0_ref[...] = cc * lg
    c2b_ref[...] = jnp.broadcast_to((1.0 - ALPHA) * cc * cc, y0_ref.shape)
    clb_ref[...] = ALPHA * cc * lg
    cb_ref[...] = jnp.broadcast_to(cc, y0_ref.shape)


def _prep(deg2, logits_pad):
    RB = 512
    return pl.pallas_call(
        _prep_body,
        grid=(NP // RB,),
        in_specs=[
            pl.BlockSpec((2, RB, 1), lambda i: (0, i, 0)),
            pl.BlockSpec((RB, C), lambda i: (i, 0)),
        ],
        out_specs=[pl.BlockSpec((RB, C), lambda i: (i, 0))] * 4,
        out_shape=[jax.ShapeDtypeStruct((NP, C), jnp.float32)] * 4,
    )(deg2.reshape(NC, NP, 1), logits_pad)


def _combine_body(acc_ref, y_ref, c2b_ref, clb_ref, out_ref):
    s = acc_ref[0] + acc_ref[1] + y_ref[...]
    out_ref[...] = c2b_ref[...] * s + clb_ref[...]


def _combine(acc, y, c2b, clb):
    RB = 512
    return pl.pallas_call(
        _combine_body,
        grid=(NP // RB,),
        in_specs=[
            pl.BlockSpec((2, RB, C), lambda i: (0, i, 0)),
            pl.BlockSpec((RB, C), lambda i: (i, 0)),
            pl.BlockSpec((RB, C), lambda i: (i, 0)),
            pl.BlockSpec((RB, C), lambda i: (i, 0)),
        ],
        out_specs=pl.BlockSpec((RB, C), lambda i: (i, 0)),
        out_shape=jax.ShapeDtypeStruct((NP, C), jnp.float32),
    )(acc, y, c2b, clb)


def _final_body(acc_ref, y_ref, cb_ref, logits_ref, out_ref):
    s = acc_ref[0] + acc_ref[1] + y_ref[...]
    out_ref[...] = (1.0 - ALPHA) * cb_ref[...] * s + ALPHA * logits_ref[...]


def _final(acc, y, cb, logits_pad):
    RB = 512
    return pl.pallas_call(
        _final_body,
        grid=(NP // RB,),
        in_specs=[
            pl.BlockSpec((2, RB, C), lambda i: (0, i, 0)),
            pl.BlockSpec((RB, C), lambda i: (i, 0)),
            pl.BlockSpec((RB, C), lambda i: (i, 0)),
            pl.BlockSpec((RB, C), lambda i: (i, 0)),
        ],
        out_specs=pl.BlockSpec((RB, C), lambda i: (i, 0)),
        out_shape=jax.ShapeDtypeStruct((NP, C), jnp.float32),
    )(acc, y, cb, logits_pad)


def kernel(features, edge_idx, W0, b0, W1, b1, W2, b2):
    src = edge_idx[0].astype(jnp.int32)
    dst = edge_idx[1].astype(jnp.int32)
    pad = jnp.full((EP - E,), N, jnp.int32)
    srcm = jnp.concatenate([src, pad]).reshape(ER, CHUNK)
    dstm = jnp.concatenate([dst, pad]).reshape(ER, CHUNK)

    deg2, srcp = _deg_kernel(srcm, dstm)
    logits = _mlp(features, W0, b0, W1, b1, W2, b2)
    logits_pad = jnp.pad(logits, ((0, NP - N), (0, 0)))
    y, c2b, clb, cb = _prep(deg2, logits_pad)

    acc = None
    for k in range(K_PROP):
        acc = _prop_kernel(y, srcp, dstm)
        if k < K_PROP - 1:
            y = _combine(acc, y, c2b, clb)
    x_pad = _final(acc, y, cb, logits_pad)
    return x_pad[:N]


# speedup vs baseline: 6.0416x; 1.1467x over previous
import jax
import jax.numpy as jnp
from jax import lax
from jax.experimental import pallas as pl
from jax.experimental.pallas import tpu as pltpu
from jax.experimental.pallas import tpu_sc as plsc

N = 10000
C = 128
E = 160000
K_PROP = 10
ALPHA = 0.1

NP = 10240
NC, NS, L = 2, 16, 16
NT = NC * NS
EP = 163840
EPT = EP // NT
CHUNK = 128
NCH = EPT // CHUNK
ER = EP // CHUNK
RPT = NP // NS

_MESH = plsc.VectorSubcoreMesh(
    core_axis_name="c", subcore_axis_name="s", num_cores=NC, num_subcores=NS
)


def _deg_body(srcm, dstm, deg_out, srcp_out, srcv, dstv, srcpv, onesv, degv,
              deg_sh):
    c = lax.axis_index("c")
    s = lax.axis_index("s")
    wid = c * NS + s
    base = wid * NCH

    pltpu.sync_copy(srcm.at[pl.ds(base, NCH)], srcv)
    pltpu.sync_copy(dstm.at[pl.ds(base, NCH)], dstv)

    for r in range(NCH):
        for q in range(CHUNK // L):
            sl = pl.ds(q * L, L)
            sv = srcv[r, sl]
            dv = dstv[r, sl]
            srcpv[r, sl] = jnp.where(sv == dv, jnp.int32(N), sv)
    pltpu.sync_copy(srcpv, srcp_out.at[pl.ds(base, NCH)])

    ones = jnp.ones((L,), jnp.float32)
    init = jnp.where(c == 0, 1.0, 0.0) * ones
    for q in range(CHUNK // L):
        onesv[pl.ds(q * L, L)] = ones
    for r in range(RPT // L):
        degv[pl.ds(r * L, L)] = init
    pltpu.sync_copy(degv, deg_sh.at[pl.ds(s * RPT, RPT)])
    plsc.subcore_barrier()

    for j in range(NCH):
        pltpu.sync_copy(onesv, deg_sh.at[srcpv.at[j]], add=True)
    plsc.subcore_barrier()

    pltpu.sync_copy(deg_sh.at[pl.ds(s * RPT, RPT)],
                    deg_out.at[c, pl.ds(s * RPT, RPT)])


_deg_kernel = pl.kernel(
    _deg_body,
    out_type=(
        jax.ShapeDtypeStruct((NC, NP), jnp.float32),
        jax.ShapeDtypeStruct((ER, CHUNK), jnp.int32),
    ),
    mesh=_MESH,
    scratch_types=(
        pltpu.VMEM((NCH, CHUNK), jnp.int32),
        pltpu.VMEM((NCH, CHUNK), jnp.int32),
        pltpu.VMEM((NCH, CHUNK), jnp.int32),
        pltpu.VMEM((CHUNK,), jnp.float32),
        pltpu.VMEM((RPT,), jnp.float32),
        pltpu.VMEM_SHARED((NP,), jnp.float32),
    ),
)


NBUF = 2
ZR = 16


def _prop_body(y_hbm, srcp, dstm, acc_out, srcv, dstv, rows, zbuf, gsems,
               acc_sh):
    c = lax.axis_index("c")
    s = lax.axis_index("s")
    wid = c * NS + s
    base = wid * NCH

    isem = gsems[NBUF]
    zsem = gsems[NBUF + 1]
    idx_a = pltpu.async_copy(srcp.at[pl.ds(base, NCH)], srcv, isem)
    idx_b = pltpu.async_copy(dstm.at[pl.ds(base, NCH)], dstv, isem)

    zv = jnp.zeros((L,), jnp.float32)
    for r in range(ZR):
        for q in range(C // L):
            zbuf[r, pl.ds(q * L, L)] = zv
    zeros = [
        pltpu.async_copy(zbuf, acc_sh.at[pl.ds(s * RPT + t * ZR, ZR)], zsem)
        for t in range(RPT // ZR)
    ]
    idx_a.wait()
    idx_b.wait()
    for z in zeros:
        z.wait()
    plsc.subcore_barrier()

    gathers = [None] * NBUF
    for j in range(NBUF):
        gathers[j % NBUF] = pltpu.async_copy(
            y_hbm.at[srcv.at[j]], rows.at[j % NBUF], gsems[j % NBUF])
    for j in range(NCH):
        b = j % NBUF
        gathers[b].wait()
        pltpu.sync_copy(rows.at[b], acc_sh.at[dstv.at[j]], add=True)
        if j + NBUF < NCH:
            gathers[b] = pltpu.async_copy(
                y_hbm.at[srcv.at[j + NBUF]], rows.at[b], gsems[b])
    plsc.subcore_barrier()

    pltpu.sync_copy(acc_sh.at[pl.ds(s * RPT, RPT)],
                    acc_out.at[c, pl.ds(s * RPT, RPT)])


_prop_kernel = pl.kernel(
    _prop_body,
    out_type=jax.ShapeDtypeStruct((NC, NP, C), jnp.float32),
    mesh=_MESH,
    scratch_types=(
        pltpu.VMEM((NCH, CHUNK), jnp.int32),
        pltpu.VMEM((NCH, CHUNK), jnp.int32),
        pltpu.VMEM((NBUF, CHUNK, C), jnp.float32),
        pltpu.VMEM((ZR, C), jnp.float32),
        (pltpu.SemaphoreType.DMA,) * (NBUF + 2),
        pltpu.VMEM_SHARED((NP, C), jnp.float32),
    ),
)


def _mlp_body(x_ref, w0_ref, b0_ref, w1_ref, b1_ref, w2_ref, b2_ref, out_ref):
    x = x_ref[...]
    h = lax.dot_general(x, w0_ref[...], (((1,), (1,)), ((), ())),
                        preferred_element_type=jnp.float32) + b0_ref[...]
    h = jnp.maximum(h, 0.0)
    h = lax.dot_general(h, w1_ref[...], (((1,), (1,)), ((), ())),
                        preferred_element_type=jnp.float32) + b1_ref[...]
    h = jnp.maximum(h, 0.0)
    out_ref[...] = lax.dot_general(h, w2_ref[...], (((1,), (1,)), ((), ())),
                                   preferred_element_type=jnp.float32) + b2_ref[...]


def _mlp(features, W0, b0, W1, b1, W2, b2):
    RB = 400
    full = lambda shape: pl.BlockSpec(shape, lambda i: (0,) * len(shape))
    return pl.pallas_call(
        _mlp_body,
        grid=(N // RB,),
        in_specs=[
            pl.BlockSpec((RB, 256), lambda i: (i, 0)),
            full((512, 256)), full((1, 512)),
            full((512, 512)), full((1, 512)),
            full((128, 512)), full((1, 128)),
        ],
        out_specs=pl.BlockSpec((RB, C), lambda i: (i, 0)),
        out_shape=jax.ShapeDtypeStruct((N, C), jnp.float32),
    )(features, W0, b0.reshape(1, 512), W1, b1.reshape(1, 512),
      W2, b2.reshape(1, 128))


def _prep_body(deg_ref, logits_ref, y0_ref, c2b_ref, clb_ref, cb_ref):
    d = deg_ref[0] + deg_ref[1]
    pid = pl.program_id(0)
    row = lax.broadcasted_iota(jnp.int32, d.shape, 0) + pid * d.shape[0]
    cc = jnp.where(row < N, lax.rsqrt(jnp.maximum(d, 1e-20)), 0.0)
    lg = logits_ref[...]
    y0_ref[...] = cc * lg
    c2b_ref[...] = jnp.broadcast_to((1.0 - ALPHA) * cc * cc, y0_ref.shape)
    clb_ref[...] = ALPHA * cc * lg
    cb_ref[...] = jnp.broadcast_to(cc, y0_ref.shape)


def _prep(deg2, logits_pad):
    RB = 512
    return pl.pallas_call(
        _prep_body,
        grid=(NP // RB,),
        in_specs=[
            pl.BlockSpec((2, RB, 1), lambda i: (0, i, 0)),
            pl.BlockSpec((RB, C), lambda i: (i, 0)),
        ],
        out_specs=[pl.BlockSpec((RB, C), lambda i: (i, 0))] * 4,
        out_shape=[jax.ShapeDtypeStruct((NP, C), jnp.float32)] * 4,
    )(deg2.reshape(NC, NP, 1), logits_pad)


def _combine_body(acc_ref, y_ref, c2b_ref, clb_ref, out_ref):
    s = acc_ref[0] + acc_ref[1] + y_ref[...]
    out_ref[...] = c2b_ref[...] * s + clb_ref[...]


def _combine(acc, y, c2b, clb):
    RB = 512
    return pl.pallas_call(
        _combine_body,
        grid=(NP // RB,),
        in_specs=[
            pl.BlockSpec((2, RB, C), lambda i: (0, i, 0)),
            pl.BlockSpec((RB, C), lambda i: (i, 0)),
            pl.BlockSpec((RB, C), lambda i: (i, 0)),
            pl.BlockSpec((RB, C), lambda i: (i, 0)),
        ],
        out_specs=pl.BlockSpec((RB, C), lambda i: (i, 0)),
        out_shape=jax.ShapeDtypeStruct((NP, C), jnp.float32),
    )(acc, y, c2b, clb)


def _final_body(acc_ref, y_ref, cb_ref, logits_ref, out_ref):
    s = acc_ref[0] + acc_ref[1] + y_ref[...]
    out_ref[...] = (1.0 - ALPHA) * cb_ref[...] * s + ALPHA * logits_ref[...]


def _final(acc, y, cb, logits_pad):
    RB = 512
    return pl.pallas_call(
        _final_body,
        grid=(NP // RB,),
        in_specs=[
            pl.BlockSpec((2, RB, C), lambda i: (0, i, 0)),
            pl.BlockSpec((RB, C), lambda i: (i, 0)),
            pl.BlockSpec((RB, C), lambda i: (i, 0)),
            pl.BlockSpec((RB, C), lambda i: (i, 0)),
        ],
        out_specs=pl.BlockSpec((RB, C), lambda i: (i, 0)),
        out_shape=jax.ShapeDtypeStruct((NP, C), jnp.float32),
    )(acc, y, cb, logits_pad)


def kernel(features, edge_idx, W0, b0, W1, b1, W2, b2):
    src = edge_idx[0].astype(jnp.int32)
    dst = edge_idx[1].astype(jnp.int32)
    pad = jnp.full((EP - E,), N, jnp.int32)
    srcm = jnp.concatenate([src, pad]).reshape(ER, CHUNK)
    dstm = jnp.concatenate([dst, pad]).reshape(ER, CHUNK)

    deg2, srcp = _deg_kernel(srcm, dstm)
    logits = _mlp(features, W0, b0, W1, b1, W2, b2)
    logits_pad = jnp.pad(logits, ((0, NP - N), (0, 0)))
    y, c2b, clb, cb = _prep(deg2, logits_pad)

    acc = None
    for k in range(K_PROP):
        acc = _prop_kernel(y, srcp, dstm)
        if k < K_PROP - 1:
            y = _combine(acc, y, c2b, clb)
    x_pad = _final(acc, y, cb, logits_pad)
    return x_pad[:N]


# speedup vs baseline: 7.2365x; 1.1978x over previous
import jax
import jax.numpy as jnp
from jax import lax
from jax.experimental import pallas as pl
from jax.experimental.pallas import tpu as pltpu
from jax.experimental.pallas import tpu_sc as plsc

N = 10000
C = 128
CH = C // 2
E = 160000
K_PROP = 10
ALPHA = 0.1

NP = 10240
NC, NS, L = 2, 16, 16
EP = 163840
EPT = EP // NS
CHUNK = 128
NCH = EPT // CHUNK
ER = EP // CHUNK
RPT = NP // NS
NBUF = 3
ZR = 16

_MESH = plsc.VectorSubcoreMesh(
    core_axis_name="c", subcore_axis_name="s", num_cores=NC, num_subcores=NS
)


DEG_RB = 8
DEG_ROWS = ER // (NC * NS)


def _deg_body(srcm, dstm, deg_out, srcp_out, srcv, dstv, srcpv, onesv, degv,
              deg_sh):
    c = lax.axis_index("c")
    s = lax.axis_index("s")
    wid = c * NS + s
    base = wid * DEG_ROWS

    ones = jnp.ones((L,), jnp.float32)
    init = jnp.where(c == 0, 1.0, 0.0) * ones
    for q in range(CHUNK // L):
        onesv[pl.ds(q * L, L)] = ones
    for r in range(RPT // L):
        degv[pl.ds(r * L, L)] = init
    pltpu.sync_copy(degv, deg_sh.at[pl.ds(s * RPT, RPT)])

    for t in range(DEG_ROWS // DEG_RB):
        rb = base + t * DEG_RB
        pltpu.sync_copy(srcm.at[pl.ds(rb, DEG_RB)], srcv)
        pltpu.sync_copy(dstm.at[pl.ds(rb, DEG_RB)], dstv)
        for r in range(DEG_RB):
            for q in range(CHUNK // L):
                sl = pl.ds(q * L, L)
                sv = srcv[r, sl]
                dv = dstv[r, sl]
                srcpv[r, sl] = jnp.where(sv == dv, jnp.int32(N), sv)
        pltpu.sync_copy(srcpv, srcp_out.at[pl.ds(rb, DEG_RB)])
    plsc.subcore_barrier()

    for t in range(DEG_ROWS // DEG_RB):
        rb = base + t * DEG_RB
        pltpu.sync_copy(srcp_out.at[pl.ds(rb, DEG_RB)], srcpv)
        for j in range(DEG_RB):
            pltpu.sync_copy(onesv, deg_sh.at[srcpv.at[j]], add=True)
    plsc.subcore_barrier()

    pltpu.sync_copy(deg_sh.at[pl.ds(s * RPT, RPT)],
                    deg_out.at[c, pl.ds(s * RPT, RPT)])


_deg_kernel = pl.kernel(
    _deg_body,
    out_type=(
        jax.ShapeDtypeStruct((NC, NP), jnp.float32),
        jax.ShapeDtypeStruct((ER, CHUNK), jnp.int32),
    ),
    mesh=_MESH,
    scratch_types=(
        pltpu.VMEM((DEG_RB, CHUNK), jnp.int32),
        pltpu.VMEM((DEG_RB, CHUNK), jnp.int32),
        pltpu.VMEM((DEG_RB, CHUNK), jnp.int32),
        pltpu.VMEM((CHUNK,), jnp.float32),
        pltpu.VMEM((RPT,), jnp.float32),
        pltpu.VMEM_SHARED((NP,), jnp.float32),
    ),
)


IB = 8
NG = NCH // IB


def _prop_body(y_hbm, srcp, dstm, acc_out, srcv, dstv, rows, zbuf, gsems,
               acc_sh):
    c = lax.axis_index("c")
    s = lax.axis_index("s")
    base = s * NCH

    isem = gsems[NBUF]
    zsem = gsems[NBUF + 1]
    y_half = y_hbm.at[c]
    idx_a = pltpu.async_copy(srcp.at[pl.ds(base, IB)], srcv.at[0], isem)
    idx_b = pltpu.async_copy(dstm.at[pl.ds(base, IB)], dstv.at[0], isem)

    zv = jnp.zeros((L,), jnp.float32)
    for r in range(ZR):
        for q in range(CH // L):
            zbuf[r, pl.ds(q * L, L)] = zv
    zeros = [
        pltpu.async_copy(zbuf, acc_sh.at[pl.ds(s * RPT + t * ZR, ZR)], zsem)
        for t in range(RPT // ZR)
    ]
    idx_a.wait()
    idx_b.wait()
    for z in zeros:
        z.wait()
    plsc.subcore_barrier()

    gathers = [None] * NBUF
    for j in range(NBUF):
        gathers[j] = pltpu.async_copy(
            y_half.at[srcv.at[0, j]], rows.at[j], gsems[j])
    for g in range(NG):
        sl = g % 2
        nsl = 1 - sl
        if g + 1 < NG:
            pa = pltpu.async_copy(
                srcp.at[pl.ds(base + (g + 1) * IB, IB)], srcv.at[nsl], isem)
            pb = pltpu.async_copy(
                dstm.at[pl.ds(base + (g + 1) * IB, IB)], dstv.at[nsl], isem)
        for jj in range(IB):
            j = g * IB + jj
            b = j % NBUF
            if g + 1 < NG and jj == IB - NBUF:
                pa.wait()
                pb.wait()
            gathers[b].wait()
            pltpu.sync_copy(rows.at[b], acc_sh.at[dstv.at[sl, jj]], add=True)
            nj = j + NBUF
            if nj < NCH:
                njj = nj % IB
                njsl = sl if nj // IB == g else nsl
                gathers[b] = pltpu.async_copy(
                    y_half.at[srcv.at[njsl, njj]], rows.at[b], gsems[b])
    plsc.subcore_barrier()

    pltpu.sync_copy(acc_sh.at[pl.ds(s * RPT, RPT)],
                    acc_out.at[c, pl.ds(s * RPT, RPT)])


_prop_kernel = pl.kernel(
    _prop_body,
    out_type=jax.ShapeDtypeStruct((NC, NP, CH), jnp.float32),
    mesh=_MESH,
    scratch_types=(
        pltpu.VMEM((2, IB, CHUNK), jnp.int32),
        pltpu.VMEM((2, IB, CHUNK), jnp.int32),
        pltpu.VMEM((NBUF, CHUNK, CH), jnp.float32),
        pltpu.VMEM((ZR, CH), jnp.float32),
        (pltpu.SemaphoreType.DMA,) * (NBUF + 2),
        pltpu.VMEM_SHARED((NP, CH), jnp.float32),
    ),
    compiler_params=pltpu.CompilerParams(use_tc_tiling_on_sc=False),
)


def _mlp_body(x_ref, w0_ref, b0_ref, w1_ref, b1_ref, w2_ref, b2_ref, out_ref):
    x = x_ref[...]
    h = lax.dot_general(x, w0_ref[...], (((1,), (1,)), ((), ())),
                        preferred_element_type=jnp.float32) + b0_ref[...]
    h = jnp.maximum(h, 0.0)
    h = lax.dot_general(h, w1_ref[...], (((1,), (1,)), ((), ())),
                        preferred_element_type=jnp.float32) + b1_ref[...]
    h = jnp.maximum(h, 0.0)
    out_ref[...] = lax.dot_general(h, w2_ref[...], (((1,), (1,)), ((), ())),
                                   preferred_element_type=jnp.float32) + b2_ref[...]


def _mlp(features, W0, b0, W1, b1, W2, b2):
    RB = 400
    full = lambda shape: pl.BlockSpec(shape, lambda i: (0,) * len(shape))
    return pl.pallas_call(
        _mlp_body,
        grid=(N // RB,),
        in_specs=[
            pl.BlockSpec((RB, 256), lambda i: (i, 0)),
            full((512, 256)), full((1, 512)),
            full((512, 512)), full((1, 512)),
            full((128, 512)), full((1, 128)),
        ],
        out_specs=pl.BlockSpec((RB, C), lambda i: (i, 0)),
        out_shape=jax.ShapeDtypeStruct((N, C), jnp.float32),
    )(features, W0, b0.reshape(1, 512), W1, b1.reshape(1, 512),
      W2, b2.reshape(1, 128))


def _prep_body(deg_ref, logits_ref, y0_ref, c2b_ref, clb_ref, cb_ref):
    d = deg_ref[0] + deg_ref[1]
    pid = pl.program_id(1)
    row = lax.broadcasted_iota(jnp.int32, d.shape, 0) + pid * d.shape[0]
    cc = jnp.where(row < N, lax.rsqrt(jnp.maximum(d, 1e-20)), 0.0)
    lg = logits_ref[0]
    y0_ref[0] = cc * lg
    c2b_ref[0] = jnp.broadcast_to((1.0 - ALPHA) * cc * cc, lg.shape)
    clb_ref[0] = ALPHA * cc * lg
    cb_ref[0] = jnp.broadcast_to(cc, lg.shape)


def _prep(deg2, logits_h):
    RB = 512
    half = [jax.ShapeDtypeStruct((NC, NP, CH), jnp.float32)] * 4
    return pl.pallas_call(
        _prep_body,
        grid=(NC, NP // RB),
        in_specs=[
            pl.BlockSpec((2, RB, 1), lambda h, i: (0, i, 0)),
            pl.BlockSpec((1, RB, CH), lambda h, i: (h, i, 0)),
        ],
        out_specs=[pl.BlockSpec((1, RB, CH), lambda h, i: (h, i, 0))] * 4,
        out_shape=half,
    )(deg2.reshape(NC, NP, 1), logits_h)


def _combine_body(acc_ref, y_ref, c2b_ref, clb_ref, out_ref):
    s = acc_ref[...] + y_ref[...]
    out_ref[...] = c2b_ref[...] * s + clb_ref[...]


def _combine(acc, y, c2b, clb):
    RB = 512
    spec = pl.BlockSpec((1, RB, CH), lambda h, i: (h, i, 0))
    return pl.pallas_call(
        _combine_body,
        grid=(NC, NP // RB),
        in_specs=[spec, spec, spec, spec],
        out_specs=spec,
        out_shape=jax.ShapeDtypeStruct((NC, NP, CH), jnp.float32),
    )(acc, y, c2b, clb)


def _final_body(acc_ref, y_ref, cb_ref, logits_ref, out_ref):
    s = acc_ref[0] + y_ref[0]
    out_ref[0] = (1.0 - ALPHA) * cb_ref[0] * s + ALPHA * logits_ref[0]


def _final(acc, y, cb, logits_h):
    RB = 512
    spec = pl.BlockSpec((1, RB, CH), lambda h, i: (h, i, 0))
    return pl.pallas_call(
        _final_body,
        grid=(NC, NP // RB),
        in_specs=[spec, spec, spec, spec],
        out_specs=spec,
        out_shape=jax.ShapeDtypeStruct((NC, NP, CH), jnp.float32),
    )(acc, y, cb, logits_h)


def kernel(features, edge_idx, W0, b0, W1, b1, W2, b2):
    src = edge_idx[0].astype(jnp.int32)
    dst = edge_idx[1].astype(jnp.int32)
    pad = jnp.full((EP - E,), N, jnp.int32)
    srcm = jnp.concatenate([src, pad]).reshape(ER, CHUNK)
    dstm = jnp.concatenate([dst, pad]).reshape(ER, CHUNK)

    deg2, srcp = _deg_kernel(srcm, dstm)
    logits = _mlp(features, W0, b0, W1, b1, W2, b2)
    logits_pad = jnp.pad(logits, ((0, NP - N), (0, 0)))
    logits_h = jnp.moveaxis(logits_pad.reshape(NP, NC, CH), 1, 0)
    y, c2b, clb, cb = _prep(deg2, logits_h)

    acc = None
    for k in range(K_PROP):
        acc = _prop_kernel(y, srcp, dstm)
        if k < K_PROP - 1:
            y = _combine(acc, y, c2b, clb)
    x_h = _final(acc, y, cb, logits_h)
    x_pad = jnp.moveaxis(x_h, 0, 1).reshape(NP, C)
    return x_pad[:N]
